# Initial kernel scaffold; baseline (speedup 1.0000x reference)
import jax
import jax.numpy as jnp
from jax.experimental import pallas as pl


def kernel(x, edge_index, W1, b1, Wmu, bmu, Wls, bls):
    raise NotImplementedError("write your pallas kernel here")


# speedup vs baseline: 9.5594x; 9.5594x over previous
import functools

import jax
import jax.numpy as jnp
from jax import lax
from jax.experimental import pallas as pl
from jax.experimental.pallas import tpu as pltpu
from jax.experimental.pallas import tpu_sc as plsc

N = 10000
E = 160000
NP = 10240
HALF = 128
TILES = 16
RPT = NP // TILES
CHUNK = 128
CH = 84
EPT = CH * CHUNK
JUNK = N + 8

_mesh = plsc.VectorSubcoreMesh(core_axis_name="c", subcore_axis_name="s")


@functools.partial(
    pl.kernel,
    out_type=jax.ShapeDtypeStruct((NP,), jnp.float32),
    mesh=_mesh,
    scratch_types=[
        pltpu.VMEM((CH, CHUNK), jnp.int32),
        pltpu.VMEM((CHUNK,), jnp.float32),
        pltpu.VMEM((RPT,), jnp.float32),
        pltpu.VMEM_SHARED((NP,), jnp.float32),
        pltpu.SemaphoreType.DMA,
    ],
)
def _sc_degree(col_hbm, deg_hbm, col_v, ones_v, zero_v, acc, sem):
    c = lax.axis_index("c")
    s = lax.axis_index("s")
    pltpu.async_copy(col_hbm.at[s], col_v, sem).wait()
    for j in range(CHUNK // 16):
        ones_v[pl.ds(j * 16, 16)] = jnp.ones((16,), jnp.float32)

    @pl.loop(0, RPT // 16)
    def _zero(i):
        zero_v[pl.ds(i * 16, 16)] = jnp.zeros((16,), jnp.float32)

    pltpu.sync_copy(zero_v, acc.at[pl.ds(s * RPT, RPT)])
    plsc.subcore_barrier()

    @pl.loop(0, CH)
    def _scatter(j):
        pltpu.sync_copy(ones_v, acc.at[col_v.at[j]], add=True)

    plsc.subcore_barrier()

    @pl.when(c == 0)
    def _writeback():
        pltpu.sync_copy(acc.at[pl.ds(s * RPT, RPT)],
                        deg_hbm.at[pl.ds(s * RPT, RPT)])


@functools.partial(
    pl.kernel,
    out_type=jax.ShapeDtypeStruct((2, NP, HALF), jnp.float32),
    mesh=_mesh,
    scratch_types=[
        pltpu.VMEM((CH, CHUNK), jnp.int32),
        pltpu.VMEM((CH, CHUNK), jnp.int32),
        pltpu.VMEM((CHUNK, HALF), jnp.float32),
        pltpu.VMEM_SHARED((NP, HALF), jnp.float32),
        pltpu.SemaphoreType.DMA,
        pltpu.SemaphoreType.DMA,
        pltpu.SemaphoreType.DMA,
    ],
)
def _sc_prop(src_hbm, rowidx_hbm, colidx_hbm, out_hbm,
             row_v, col_v, dbuf, acc, gsem, ssem, isem):
    c = lax.axis_index("c")
    s = lax.axis_index("s")
    pltpu.async_copy(rowidx_hbm.at[c, s], row_v, isem).wait()
    pltpu.async_copy(colidx_hbm.at[s], col_v, isem).wait()

    @pl.loop(0, CHUNK)
    def _zfill(i):
        for j in range(HALF // 16):
            dbuf[i, pl.ds(j * 16, 16)] = jnp.zeros((16,), jnp.float32)

    for k in range(RPT // CHUNK):
        pltpu.sync_copy(dbuf, acc.at[pl.ds(s * RPT + k * CHUNK, CHUNK)])
    plsc.subcore_barrier()

    @pl.loop(0, CH)
    def _edges(j):
        pltpu.async_copy(src_hbm.at[row_v.at[j]], dbuf, gsem).wait()
        pltpu.async_copy(dbuf, acc.at[col_v.at[j]], ssem, add=True).wait()

    plsc.subcore_barrier()
    pltpu.sync_copy(acc.at[pl.ds(s * RPT, RPT)],
                    out_hbm.at[c, pl.ds(s * RPT, RPT)])


def _tc_pre_body(x_ref, deg_ref, dinv_ref, xs_ref):
    dinv = lax.rsqrt(jnp.maximum(deg_ref[...], 1.0))
    dinv_ref[...] = dinv
    xs = x_ref[...] * dinv[:, None]
    xs_ref[0] = xs[:, :HALF]
    xs_ref[1] = xs[:, HALF:]


_tc_pre = pl.pallas_call(
    _tc_pre_body,
    out_shape=[
        jax.ShapeDtypeStruct((NP,), jnp.float32),
        jax.ShapeDtypeStruct((2, NP, HALF), jnp.float32),
    ],
)


def _tc_mid_body(s1_ref, dinv_ref, w1_ref, b1_ref, hs_ref):
    dinv = dinv_ref[...]
    agg = jnp.concatenate([s1_ref[0], s1_ref[1]], axis=1) * dinv[:, None]
    h = jnp.maximum(
        jnp.dot(agg, w1_ref[...], preferred_element_type=jnp.float32)
        + b1_ref[...][None, :], 0.0)
    hs = h * dinv[:, None]
    hs_ref[0] = hs[:, :HALF]
    hs_ref[1] = hs[:, HALF:]


_tc_mid = pl.pallas_call(
    _tc_mid_body,
    out_shape=jax.ShapeDtypeStruct((2, NP, HALF), jnp.float32),
)


def _tc_fin_body(s2_ref, dinv_ref, wmu_ref, bmu_ref, wls_ref, bls_ref,
                 mu_ref, ls_ref):
    dinv = dinv_ref[...]
    agg = jnp.concatenate([s2_ref[0], s2_ref[1]], axis=1) * dinv[:, None]
    mu_ref[...] = (jnp.dot(agg, wmu_ref[...],
                           preferred_element_type=jnp.float32)
                   + bmu_ref[...][None, :])
    ls_ref[...] = (jnp.dot(agg, wls_ref[...],
                           preferred_element_type=jnp.float32)
                   + bls_ref[...][None, :])


_tc_fin = pl.pallas_call(
    _tc_fin_body,
    out_shape=[
        jax.ShapeDtypeStruct((NP, HALF), jnp.float32),
        jax.ShapeDtypeStruct((NP, HALF), jnp.float32),
    ],
)


def kernel(x, edge_index, W1, b1, Wmu, bmu, Wls, bls):
    ei = edge_index.astype(jnp.int32)
    loop = jnp.arange(N, dtype=jnp.int32)
    row_a = jnp.concatenate([ei[0], loop])
    col_a = jnp.concatenate([ei[1], loop])
    pad = EPT * TILES - row_a.shape[0]
    row_p = jnp.concatenate([row_a, jnp.full((pad,), N, jnp.int32)])
    col_p = jnp.concatenate([col_a, jnp.full((pad,), JUNK, jnp.int32)])
    rowidx = jnp.stack([row_p, row_p + NP]).reshape(2, TILES, CH, CHUNK)
    colidx = col_p.reshape(TILES, CH, CHUNK)
    x_pad = jnp.pad(x, ((0, NP - N), (0, 0)))

    deg = _sc_degree(colidx)
    dinv, xs = _tc_pre(x_pad, deg)
    s1 = _sc_prop(xs.reshape(2 * NP, HALF), rowidx, colidx)
    hs = _tc_mid(s1, dinv, W1, b1)
    s2 = _sc_prop(hs.reshape(2 * NP, HALF), rowidx, colidx)
    mu, ls = _tc_fin(s2, dinv, Wmu, bmu, Wls, bls)
    return mu[:N], ls[:N]
